# baseline (speedup 1.0000x reference)
import functools

import jax
import jax.numpy as jnp
from jax import lax
from jax.experimental import pallas as pl
from jax.experimental.pallas import tpu as pltpu
from jax.experimental.pallas import tpu_sc as plsc

B = 32
LQ = 32
LK = 2048
D = 1024
N_HEAD = 16
D_H = 64
FILT = 10
KSPLIT = 8
KC = D // KSPLIT
L = 16


def _scores_topk_kernel(q0_ref, k_ref, idx_ref, scores_ref):
    g = pl.program_id(0)
    prev_b = jnp.clip(g - 1, 0, B - 1)

    s = scores_ref[...]
    lin = (jax.lax.broadcasted_iota(jnp.int32, (16, 128), 0) * 128
           + jax.lax.broadcasted_iota(jnp.int32, (16, 128), 1))
    lane16 = jax.lax.broadcasted_iota(jnp.int32, (1, L), 1)
    picks = jnp.zeros((1, L), jnp.int32)
    for j in range(FILT):
        m = jnp.max(s)
        idx = jnp.min(jnp.where(s >= m, lin, jnp.int32(LK)))
        idx = jnp.minimum(idx, LK - 1)
        s = jnp.where(lin == idx, -jnp.inf, s)
        picks = jnp.where(lane16 == j, idx, picks)
    idx_ref[0] = picks + prev_b * LK

    q0_bf = q0_ref[0].astype(jnp.bfloat16)
    kb_bf = k_ref[0].astype(jnp.bfloat16)
    parts = []
    for c in range(KSPLIT):
        ksl = slice(c * KC, (c + 1) * KC)
        parts.append(jax.lax.dot_general(
            q0_bf[:, ksl], kb_bf[:, ksl], (((1,), (1,)), ((), ())),
            preferred_element_type=jnp.float32))
    snew = parts[0]
    for c in range(1, KSPLIT):
        snew = snew + parts[c]
    scores_ref[...] = jnp.concatenate(
        [snew[:, i * 128:(i + 1) * 128] for i in range(16)], axis=0)


def _sc_gather(idx_hbm, kflat_hbm, vflat_hbm, fk_hbm, fv_hbm,
               idxv, krows, vrows, sem):
    w = lax.axis_index("s") * 2 + lax.axis_index("c")
    pltpu.sync_copy(idx_hbm.at[w], idxv)
    pltpu.async_copy(kflat_hbm.at[idxv], krows, sem).wait()
    pltpu.async_copy(vflat_hbm.at[idxv], vrows, sem).wait()
    pltpu.sync_copy(krows, fk_hbm.at[w])
    pltpu.sync_copy(vrows, fv_hbm.at[w])


def _attn_kernel(q_ref, fk_ref, fv_ref, out_ref):
    qb = q_ref[0]
    kf = fk_ref[0, 0:FILT, :]
    vf = fv_ref[0, 0:FILT, :]
    for h in range(N_HEAD):
        sl = slice(h * D_H, (h + 1) * D_H)
        qh = qb[:, sl].astype(jnp.bfloat16)
        kh = kf[:, sl].astype(jnp.bfloat16)
        vh = vf[:, sl].astype(jnp.bfloat16)
        att = jax.lax.dot_general(qh, kh, (((1,), (1,)), ((), ())),
                                  preferred_element_type=jnp.float32)
        att = att - jnp.max(att, axis=1, keepdims=True)
        e = jnp.exp(att)
        p = (e / jnp.sum(e, axis=1, keepdims=True)).astype(jnp.bfloat16)
        out_ref[0, :, sl] = jax.lax.dot_general(
            p, vh, (((1,), (0,)), ((), ())),
            preferred_element_type=jnp.float32)


def kernel(q, k, v):
    q0t = q[:, 0:1, :]

    sidx = pl.pallas_call(
        _scores_topk_kernel,
        grid=(B + 1,),
        in_specs=[
            pl.BlockSpec((1, 1, D), lambda g: (jnp.minimum(g, B - 1), 0, 0)),
            pl.BlockSpec((1, LK, D), lambda g: (jnp.minimum(g, B - 1), 0, 0)),
        ],
        out_specs=pl.BlockSpec((1, 1, L), lambda g: (jnp.maximum(g - 1, 0),
                                                     0, 0)),
        out_shape=jax.ShapeDtypeStruct((B, 1, L), jnp.int32),
        scratch_shapes=[pltpu.VMEM((16, 128), jnp.float32)],
    )(q0t, k)

    mesh = plsc.VectorSubcoreMesh(core_axis_name="c", subcore_axis_name="s")
    sc = functools.partial(
        pl.kernel, mesh=mesh,
        out_type=[
            jax.ShapeDtypeStruct((B, L, D), jnp.float32),
            jax.ShapeDtypeStruct((B, L, D), jnp.float32),
        ],
        scratch_types=[
            pltpu.VMEM((L,), jnp.int32),
            pltpu.VMEM((L, D), jnp.float32),
            pltpu.VMEM((L, D), jnp.float32),
            pltpu.SemaphoreType.DMA,
        ],
    )(_sc_gather)
    f_k, f_v = sc(sidx.reshape(B, L), k.reshape(B * LK, D),
                  v.reshape(B * LK, D))

    out = pl.pallas_call(
        _attn_kernel,
        grid=(B,),
        in_specs=[
            pl.BlockSpec((1, LQ, D), lambda b: (b, 0, 0)),
            pl.BlockSpec((1, L, D), lambda b: (b, 0, 0)),
            pl.BlockSpec((1, L, D), lambda b: (b, 0, 0)),
        ],
        out_specs=pl.BlockSpec((1, LQ, D), lambda b: (b, 0, 0)),
        out_shape=jax.ShapeDtypeStruct((B, LQ, D), jnp.float32),
    )(q, f_k, f_v)
    return out


# speedup vs baseline: 1.3334x; 1.3334x over previous
import jax
import jax.numpy as jnp
from jax.experimental import pallas as pl
from jax.experimental.pallas import tpu as pltpu

B = 32
LQ = 32
LK = 2048
D = 1024
N_HEAD = 16
D_H = 64
FILT = 10
KSPLIT = 8
KC = D // KSPLIT


def _fused_kernel(q0_ref, k_ref, q_ref, khbm_ref, vhbm_ref, out_ref,
                  scores_ref, fk_ref, fv_ref, sems):
    g = pl.program_id(0)

    @pl.when(g == 0)
    def _precharge():
        for pslot in (2, 0):
            for sz, off in ((8, 0), (1, 8), (1, 9)):
                pltpu.make_async_copy(khbm_ref.at[0, pl.ds(off, sz), :],
                                      fk_ref.at[pslot, pl.ds(off, sz), :],
                                      sems.at[pslot]).start()
                pltpu.make_async_copy(vhbm_ref.at[0, pl.ds(off, sz), :],
                                      fv_ref.at[pslot, pl.ds(off, sz), :],
                                      sems.at[pslot]).start()

    aslot = jax.lax.rem(g + 2, 3)

    for sz, off in ((8, 0), (1, 8), (1, 9)):
        pltpu.make_async_copy(khbm_ref.at[0, pl.ds(off, sz), :],
                              fk_ref.at[aslot, pl.ds(off, sz), :],
                              sems.at[aslot]).wait()
        pltpu.make_async_copy(vhbm_ref.at[0, pl.ds(off, sz), :],
                              fv_ref.at[aslot, pl.ds(off, sz), :],
                              sems.at[aslot]).wait()

    qb = q_ref[0]
    kf = fk_ref[aslot, 0:FILT, :]
    vf = fv_ref[aslot, 0:FILT, :]
    for h in range(N_HEAD):
        sl = slice(h * D_H, (h + 1) * D_H)
        qh = qb[:, sl].astype(jnp.bfloat16)
        kh = kf[:, sl].astype(jnp.bfloat16)
        vh = vf[:, sl].astype(jnp.bfloat16)
        att = jax.lax.dot_general(qh, kh, (((1,), (1,)), ((), ())),
                                  preferred_element_type=jnp.float32)
        att = att - jnp.max(att, axis=1, keepdims=True)
        e = jnp.exp(att)
        p = (e / jnp.sum(e, axis=1, keepdims=True)).astype(jnp.bfloat16)
        out_ref[0, :, sl] = jax.lax.dot_general(
            p, vh, (((1,), (0,)), ((), ())),
            preferred_element_type=jnp.float32)

    prev_b = jnp.clip(g - 1, 0, B - 1)
    slot = jax.lax.rem(g + 1, 3)

    s = scores_ref[...]
    lin = (jax.lax.broadcasted_iota(jnp.int32, (16, 128), 0) * 128
           + jax.lax.broadcasted_iota(jnp.int32, (16, 128), 1))
    for j in range(FILT):
        m = jnp.max(s)
        idx = jnp.min(jnp.where(s >= m, lin, jnp.int32(LK)))
        idx = jnp.minimum(idx, LK - 1)
        s = jnp.where(lin == idx, -jnp.inf, s)
        pltpu.make_async_copy(khbm_ref.at[prev_b, pl.ds(idx, 1), :],
                              fk_ref.at[slot, pl.ds(j, 1), :],
                              sems.at[slot]).start()
        pltpu.make_async_copy(vhbm_ref.at[prev_b, pl.ds(idx, 1), :],
                              fv_ref.at[slot, pl.ds(j, 1), :],
                              sems.at[slot]).start()

    q0_bf = q0_ref[0].astype(jnp.bfloat16)
    kb_bf = k_ref[0].astype(jnp.bfloat16)
    parts = []
    for c in range(KSPLIT):
        ksl = slice(c * KC, (c + 1) * KC)
        parts.append(jax.lax.dot_general(
            q0_bf[:, ksl], kb_bf[:, ksl], (((1,), (1,)), ((), ())),
            preferred_element_type=jnp.float32))
    snew = parts[0]
    for c in range(1, KSPLIT):
        snew = snew + parts[c]
    scores_ref[...] = jnp.concatenate(
        [snew[:, i * 128:(i + 1) * 128] for i in range(16)], axis=0)

    @pl.when(g == B + 2)
    def _final_drain():
        for dslot_shift in (0, 1):
            dslot = jax.lax.rem(g + dslot_shift, 3)
            for sz, off in ((8, 0), (1, 8), (1, 9)):
                pltpu.make_async_copy(khbm_ref.at[0, pl.ds(off, sz), :],
                                      fk_ref.at[dslot, pl.ds(off, sz), :],
                                      sems.at[dslot]).wait()
                pltpu.make_async_copy(vhbm_ref.at[0, pl.ds(off, sz), :],
                                      fv_ref.at[dslot, pl.ds(off, sz), :],
                                      sems.at[dslot]).wait()


def kernel(q, k, v):
    q0t = q[:, 0:1, :]

    out = pl.pallas_call(
        _fused_kernel,
        grid=(B + 3,),
        in_specs=[
            pl.BlockSpec((1, 1, D), lambda g: (jnp.minimum(g, B - 1), 0, 0)),
            pl.BlockSpec((1, LK, D), lambda g: (jnp.minimum(g, B - 1), 0, 0)),
            pl.BlockSpec((1, LQ, D), lambda g: (jnp.maximum(g - 3, 0), 0, 0)),
            pl.BlockSpec(memory_space=pl.ANY),
            pl.BlockSpec(memory_space=pl.ANY),
        ],
        out_specs=pl.BlockSpec((1, LQ, D), lambda g: (jnp.maximum(g - 3, 0),
                                                      0, 0)),
        out_shape=jax.ShapeDtypeStruct((B, LQ, D), jnp.float32),
        scratch_shapes=[
            pltpu.VMEM((16, 128), jnp.float32),
            pltpu.VMEM((3, 16, D), jnp.float32),
            pltpu.VMEM((3, 16, D), jnp.float32),
            pltpu.SemaphoreType.DMA((3,)),
        ],
    )(q0t, k, q, k, v)

    return out
